# Initial kernel scaffold; baseline (speedup 1.0000x reference)
import jax
import jax.numpy as jnp
from jax.experimental import pallas as pl


def kernel(features, edge_index, W_lift, b_lift, W1, b1, W2, b2, W3, b3, W_ro, b_ro):
    raise NotImplementedError("write your pallas kernel here")


# speedup vs baseline: 14.1860x; 14.1860x over previous
import jax
import jax.numpy as jnp
from jax import lax
from jax.experimental import pallas as pl

N_NODES_C = 66
N_EDGES_C = 2048
BATCH_C = 3
PER_BATCH_C = 22

_HI = lax.Precision.HIGHEST


def _gcn_kernel(src_ref, dst_ref, f_ref, wl_ref, bl_ref,
                w1_ref, b1_ref, w2_ref, b2_ref, w3_ref, b3_ref,
                wr_ref, bro_ref, out_ref):
    src = src_ref[:, :]
    dst = dst_ref[:, :]
    node_iota = lax.broadcasted_iota(jnp.int32, (N_NODES_C, N_EDGES_C), 0)
    ohs = (node_iota == src).astype(jnp.float32)
    ohd = (node_iota == dst).astype(jnp.float32)
    A = lax.dot_general(ohd, ohs, (((1,), (1,)), ((), ())),
                        preferred_element_type=jnp.float32)

    x = lax.dot_general(f_ref[:, :], wl_ref[:, :], (((1,), (1,)), ((), ())),
                        precision=_HI, preferred_element_type=jnp.float32)
    x = jnp.maximum(x + bl_ref[:, :], 0.0)

    for w_ref, b_ref in ((w1_ref, b1_ref), (w2_ref, b2_ref), (w3_ref, b3_ref)):
        agg = lax.dot_general(A, x, (((1,), (0,)), ((), ())),
                              precision=_HI, preferred_element_type=jnp.float32)
        x = lax.dot_general(agg, w_ref[:, :], (((1,), (1,)), ((), ())),
                            precision=_HI, preferred_element_type=jnp.float32)
        x = jnp.maximum(x + b_ref[:, :], 0.0)

    weighted = x * wr_ref[:, :]
    rows = jnp.sum(weighted, axis=1, keepdims=True)
    sel_n = lax.broadcasted_iota(jnp.int32, (BATCH_C, N_NODES_C), 1)
    sel_b = lax.broadcasted_iota(jnp.int32, (BATCH_C, N_NODES_C), 0)
    S = (sel_n // PER_BATCH_C == sel_b).astype(jnp.float32)
    out = lax.dot_general(S, rows, (((1,), (0,)), ((), ())),
                          precision=_HI, preferred_element_type=jnp.float32)
    out_ref[:, :] = out + bro_ref[:, :]


def kernel(features, edge_index, W_lift, b_lift, W1, b1, W2, b2, W3, b3, W_ro, b_ro):
    src = edge_index[0:1, :]
    dst = edge_index[1:2, :]
    wr_tiled = jnp.tile(W_ro.reshape(PER_BATCH_C, -1), (BATCH_C, 1))
    out = pl.pallas_call(
        _gcn_kernel,
        out_shape=jax.ShapeDtypeStruct((BATCH_C, 1), jnp.float32),
    )(src, dst, features, W_lift, b_lift.reshape(1, -1),
      W1, b1.reshape(1, -1), W2, b2.reshape(1, -1), W3, b3.reshape(1, -1),
      wr_tiled, b_ro.reshape(1, 1))
    return out
